# baseline (speedup 1.0000x reference)
import functools

import jax
import jax.numpy as jnp
from jax import lax
from jax.experimental import pallas as pl
from jax.experimental.pallas import tpu as pltpu
from jax.experimental.pallas import tpu_sc as plsc

B = 16384
DM = 32
NC = 2
NS = 16
NW = NC * NS
BPW = B // NW
CHUNK = 128
NCHUNK = BPW // CHUNK


def _pmf_body(ui_hbm, ii_hbm, u_hbm, v_hbm, c_hbm, d_hbm, out_hbm,
              uidx, iidx, ub, vb, cb, db, ob, sem):
    wid = lax.axis_index("s") * NC + lax.axis_index("c")
    base = wid * BPW

    pltpu.sync_copy(ui_hbm.at[pl.ds(base, BPW)], uidx)
    pltpu.sync_copy(ii_hbm.at[pl.ds(base, BPW)], iidx)

    copies = []
    for j in range(NCHUNK):
        us = uidx.at[pl.ds(j * CHUNK, CHUNK)]
        it = iidx.at[pl.ds(j * CHUNK, CHUNK)]
        row = pl.ds(j * CHUNK, CHUNK)
        copies.append(pltpu.async_copy(u_hbm.at[us], ub.at[row], sem))
        copies.append(pltpu.async_copy(v_hbm.at[it], vb.at[row], sem))
        copies.append(pltpu.async_copy(c_hbm.at[us], cb.at[row], sem))
        copies.append(pltpu.async_copy(d_hbm.at[it], db.at[row], sem))
    for cp in copies:
        cp.wait()

    lane = lax.iota(jnp.int32, 16)

    def group(g, _):
        rows = g * 16 + lane
        acc = jnp.zeros((16,), jnp.float32)
        for k in range(DM):
            col = (lane + k) & (DM - 1)
            pu = plsc.load_gather(ub, [rows, col])
            pv = plsc.load_gather(vb, [rows, col])
            pc = plsc.load_gather(cb, [rows, col])
            pd = plsc.load_gather(db, [rows, col])
            acc = acc + pu * pv + pc * pd
        ob[pl.ds(g * 16, 16)] = acc
        return 0

    lax.fori_loop(0, BPW // 16, group, 0)

    pltpu.sync_copy(ob, out_hbm.at[pl.ds(base, BPW)])


@jax.jit
def _pmf(users_index, items_index, U, V, C, D):
    mesh = plsc.VectorSubcoreMesh(core_axis_name="c", subcore_axis_name="s")
    f = functools.partial(
        pl.kernel,
        mesh=mesh,
        compiler_params=pltpu.CompilerParams(
            needs_layout_passes=False, use_tc_tiling_on_sc=False),
        out_type=jax.ShapeDtypeStruct((B,), jnp.float32),
        scratch_types=[
            pltpu.VMEM((BPW,), jnp.int32),
            pltpu.VMEM((BPW,), jnp.int32),
            pltpu.VMEM((BPW, DM), jnp.float32),
            pltpu.VMEM((BPW, DM), jnp.float32),
            pltpu.VMEM((BPW, DM), jnp.float32),
            pltpu.VMEM((BPW, DM), jnp.float32),
            pltpu.VMEM((BPW,), jnp.float32),
            pltpu.SemaphoreType.DMA,
        ],
    )(_pmf_body)
    return f(users_index, items_index, U, V, C, D)


def kernel(users_index, items_index, U, V, C, D):
    return _pmf(users_index.astype(jnp.int32), items_index, U, V, C, D)
